# Initial kernel scaffold; baseline (speedup 1.0000x reference)
import jax
import jax.numpy as jnp
from jax.experimental import pallas as pl


def kernel(x, W):
    raise NotImplementedError("write your pallas kernel here")


# speedup vs baseline: 1.6215x; 1.6215x over previous
import functools

import jax
import jax.numpy as jnp
from jax import lax
from jax.experimental import pallas as pl
from jax.experimental.pallas import tpu as pltpu
from jax.experimental.pallas import tpu_sc as plsc

NC = 2
NS = 16
NW = NC * NS

D = 2048
K = 32


@functools.partial(jax.jit, static_argnums=(2, 3))
def _emb_lookup(idx, table, b_per_w, nchunk):
    mesh = plsc.VectorSubcoreMesh(
        core_axis_name="c", subcore_axis_name="s",
        num_cores=NC, num_subcores=NS,
    )
    B = NW * b_per_w

    @functools.partial(
        pl.kernel,
        out_type=jax.ShapeDtypeStruct((B, D), jnp.float32),
        mesh=mesh,
        scratch_types=[
            pltpu.VMEM((nchunk, K), jnp.int32),
            pltpu.VMEM((K, D), jnp.float32),
            pltpu.SemaphoreType.DMA,
        ],
    )
    def body(idx_hbm, table_hbm, out_hbm, idx_v, rows_v, sem):
        wid = lax.axis_index("s") * NC + lax.axis_index("c")
        pltpu.sync_copy(idx_hbm.at[wid], idx_v)
        base = wid * b_per_w

        def chunk(i, carry):
            pltpu.async_copy(table_hbm.at[idx_v.at[i]], rows_v, sem).wait()
            pltpu.sync_copy(rows_v, out_hbm.at[pl.ds(base + i * K, K)])
            return carry

        lax.fori_loop(0, nchunk, chunk, 0)

    return body(idx, table)


def kernel(x, W):
    B = x.size
    b_per_w = B // NW
    nchunk = b_per_w // K
    idx = x.reshape(NW, nchunk, K)
    out = _emb_lookup(idx, W, b_per_w, nchunk)
    return out.reshape(x.shape + (W.shape[1],))


# speedup vs baseline: 1.7041x; 1.0509x over previous
import functools

import jax
import jax.numpy as jnp
from jax import lax
from jax.experimental import pallas as pl
from jax.experimental.pallas import tpu as pltpu
from jax.experimental.pallas import tpu_sc as plsc

NC = 2
NS = 16
NW = NC * NS

D = 2048
K = 16


@functools.partial(jax.jit, static_argnums=(2, 3))
def _emb_lookup(idx, table, b_per_w, nchunk):
    mesh = plsc.VectorSubcoreMesh(
        core_axis_name="c", subcore_axis_name="s",
        num_cores=NC, num_subcores=NS,
    )
    B = NW * b_per_w

    @functools.partial(
        pl.kernel,
        out_type=jax.ShapeDtypeStruct((B, D), jnp.float32),
        mesh=mesh,
        scratch_types=[
            pltpu.VMEM((nchunk, K), jnp.int32),
            pltpu.VMEM((K, D), jnp.float32),
            pltpu.VMEM((K, D), jnp.float32),
            pltpu.SemaphoreType.DMA,
            pltpu.SemaphoreType.DMA,
            pltpu.SemaphoreType.DMA,
            pltpu.SemaphoreType.DMA,
        ],
    )
    def body(idx_hbm, table_hbm, out_hbm, idx_v, rows0, rows1,
             si0, si1, so0, so1):
        wid = lax.axis_index("s") * NC + lax.axis_index("c")
        pltpu.sync_copy(idx_hbm.at[wid], idx_v)
        base = wid * b_per_w

        def gather(c, buf, sem):
            pltpu.async_copy(table_hbm.at[idx_v.at[c]], buf, sem)

        def gather_wait(buf, sem):
            pltpu.make_async_copy(table_hbm.at[idx_v.at[0]], buf, sem).wait()

        def wb(c, buf, sem):
            pltpu.async_copy(buf, out_hbm.at[pl.ds(base + c * K, K)], sem)

        def wb_wait(buf, sem):
            pltpu.make_async_copy(buf, out_hbm.at[pl.ds(base, K)], sem).wait()

        gather(0, rows0, si0)
        gather_wait(rows0, si0)
        wb(0, rows0, so0)
        gather(1, rows1, si1)

        @pl.loop(1, nchunk - 1, step=2)
        def ring(g):
            gather_wait(rows1, si1)
            wb(g, rows1, so1)
            wb_wait(rows0, so0)
            gather(g + 1, rows0, si0)
            gather_wait(rows0, si0)
            wb(g + 1, rows0, so0)
            wb_wait(rows1, so1)
            gather(g + 2, rows1, si1)

        gather_wait(rows1, si1)
        wb(nchunk - 1, rows1, so1)
        wb_wait(rows0, so0)
        wb_wait(rows1, so1)

    return body(idx, table)


def kernel(x, W):
    B = x.size
    b_per_w = B // NW
    nchunk = b_per_w // K
    idx = x.reshape(NW, nchunk, K)
    out = _emb_lookup(idx, W, b_per_w, nchunk)
    return out.reshape(x.shape + (W.shape[1],))


# speedup vs baseline: 1.7369x; 1.0193x over previous
import functools

import jax
import jax.numpy as jnp
from jax import lax
from jax.experimental import pallas as pl
from jax.experimental.pallas import tpu as pltpu
from jax.experimental.pallas import tpu_sc as plsc

NC = 2
NS = 16
NW = NC * NS

D = 2048
K = 16
NBUF = 3


@functools.partial(jax.jit, static_argnums=(2, 3))
def _emb_lookup(idx, table, b_per_w, nchunk):
    mesh = plsc.VectorSubcoreMesh(
        core_axis_name="c", subcore_axis_name="s",
        num_cores=NC, num_subcores=NS,
    )
    B = NW * b_per_w

    @functools.partial(
        pl.kernel,
        out_type=jax.ShapeDtypeStruct((B, D), jnp.float32),
        mesh=mesh,
        scratch_types=[
            pltpu.VMEM((nchunk, K), jnp.int32),
            [pltpu.VMEM((K, D), jnp.float32)] * NBUF,
            [pltpu.SemaphoreType.DMA] * NBUF,
            [pltpu.SemaphoreType.DMA] * NBUF,
        ],
    )
    def body(idx_hbm, table_hbm, out_hbm, idx_v, bufs, sin, sout):
        wid = lax.axis_index("s") * NC + lax.axis_index("c")
        pltpu.sync_copy(idx_hbm.at[wid], idx_v)
        base = wid * b_per_w

        def gather(c, b):
            pltpu.async_copy(table_hbm.at[idx_v.at[c]], bufs[b], sin[b])

        def gather_wait(b):
            pltpu.make_async_copy(
                table_hbm.at[idx_v.at[0]], bufs[b], sin[b]).wait()

        def wb(c, b):
            pltpu.async_copy(
                bufs[b], out_hbm.at[pl.ds(base + c * K, K)], sout[b])

        def wb_wait(b):
            pltpu.make_async_copy(
                bufs[b], out_hbm.at[pl.ds(base, K)], sout[b]).wait()

        def step(c, b, issue_next):
            gather_wait(b)
            wb(c, b)
            if issue_next:
                nb = (b + 1) % NBUF
                wb_wait(nb)
                gather(c + 1, nb)

        for b in range(NBUF):
            gather(b, b)
        for b in range(NBUF):
            step(b, b, b == NBUF - 1)

        nrings = nchunk // NBUF
        tail = nchunk - nrings * NBUF

        @pl.loop(1, nrings)
        def ring(r):
            for b in range(NBUF):
                step(r * NBUF + b, b, True)

        for t in range(tail):
            c = nrings * NBUF + t
            step(c, c % NBUF, c < nchunk - 1)
        for b in range(NBUF):
            wb_wait((nchunk - NBUF + b) % NBUF)

    return body(idx, table)


def kernel(x, W):
    B = x.size
    b_per_w = B // NW
    nchunk = b_per_w // K
    idx = x.reshape(NW, nchunk, K)
    out = _emb_lookup(idx, W, b_per_w, nchunk)
    return out.reshape(x.shape + (W.shape[1],))


# speedup vs baseline: 1.7506x; 1.0079x over previous
import functools

import jax
import jax.numpy as jnp
from jax import lax
from jax.experimental import pallas as pl
from jax.experimental.pallas import tpu as pltpu
from jax.experimental.pallas import tpu_sc as plsc

NC = 2
NS = 16
NW = NC * NS

D = 2048
K = 16
NBUF = 3


@functools.partial(jax.jit, static_argnums=(2, 3))
def _emb_lookup(idx, table, b_per_w, nchunk):
    mesh = plsc.VectorSubcoreMesh(
        core_axis_name="c", subcore_axis_name="s",
        num_cores=NC, num_subcores=NS,
    )
    B = NW * b_per_w

    @functools.partial(
        pl.kernel,
        out_type=jax.ShapeDtypeStruct((B, D), jnp.float32),
        mesh=mesh,
        scratch_types=[
            pltpu.VMEM((nchunk, K), jnp.int32),
            [pltpu.VMEM((K, D), jnp.float32)] * NBUF,
            [pltpu.SemaphoreType.DMA] * NBUF,
            [pltpu.SemaphoreType.DMA] * NBUF,
        ],
    )
    def body(idx_hbm, table_hbm, out_hbm, idx_v, bufs, sin, sout):
        wid = lax.axis_index("s") * NC + lax.axis_index("c")
        pltpu.sync_copy(idx_hbm.at[wid], idx_v)
        base = wid * b_per_w

        def gather(c, b):
            pltpu.async_copy(table_hbm.at[idx_v.at[c]], bufs[b], sin[b])

        def gather_wait(b):
            pltpu.make_async_copy(
                table_hbm.at[idx_v.at[0]], bufs[b], sin[b]).wait()

        def wb(c, b):
            pltpu.async_copy(
                bufs[b], out_hbm.at[pl.ds(base + c * K, K)], sout[b])

        def wb_wait(b):
            pltpu.make_async_copy(
                bufs[b], out_hbm.at[pl.ds(base, K)], sout[b]).wait()

        def step(c, b, wait_out, issue):
            gather_wait(b)
            wb(c, b)
            if wait_out:
                wb_wait((b + 2) % NBUF)
            if issue:
                gather(c + 2, (b + 2) % NBUF)

        gather(0, 0)
        gather(1, 1)
        step(0, 0, False, True)
        step(1, 1, True, True)
        step(2, 2, True, True)

        nrings = nchunk // NBUF

        @pl.loop(1, nrings)
        def ring(r):
            for b in range(NBUF):
                step(r * NBUF + b, b, True, True)

        step(nchunk - 2, (nchunk - 2) % NBUF, True, False)
        step(nchunk - 1, (nchunk - 1) % NBUF, False, False)
        wb_wait((nchunk - 2) % NBUF)
        wb_wait((nchunk - 1) % NBUF)

    return body(idx, table)


def kernel(x, W):
    B = x.size
    b_per_w = B // NW
    nchunk = b_per_w // K
    idx = x.reshape(NW, nchunk, K)
    out = _emb_lookup(idx, W, b_per_w, nchunk)
    return out.reshape(x.shape + (W.shape[1],))


# speedup vs baseline: 1.7593x; 1.0050x over previous
import functools

import jax
import jax.numpy as jnp
from jax import lax
from jax.experimental import pallas as pl
from jax.experimental.pallas import tpu as pltpu
from jax.experimental.pallas import tpu_sc as plsc

NC = 2
NS = 16
NW = NC * NS

D = 2048
K = 16
NBUF = 3


@functools.partial(jax.jit, static_argnums=(2, 3))
def _emb_lookup(idx, table, b_per_w, nchunk):
    mesh = plsc.VectorSubcoreMesh(
        core_axis_name="c", subcore_axis_name="s",
        num_cores=NC, num_subcores=NS,
    )
    B = NW * b_per_w

    @functools.partial(
        pl.kernel,
        out_type=jax.ShapeDtypeStruct((B, D), jnp.float32),
        mesh=mesh,
        scratch_types=[
            pltpu.VMEM((b_per_w,), jnp.int32),
            [pltpu.VMEM((K, D), jnp.float32)] * NBUF,
            [pltpu.SemaphoreType.DMA] * NBUF,
            [pltpu.SemaphoreType.DMA] * NBUF,
        ],
    )
    def body(idx_hbm, table_hbm, out_hbm, idx_v, bufs, sin, sout):
        wid = lax.axis_index("s") * NC + lax.axis_index("c")
        base = wid * b_per_w
        pltpu.sync_copy(idx_hbm.at[pl.ds(base, b_per_w)], idx_v)

        def gather(c, b):
            pltpu.async_copy(
                table_hbm.at[idx_v.at[pl.ds(c * K, K)]], bufs[b], sin[b])

        def gather_wait(b):
            pltpu.make_async_copy(
                table_hbm.at[idx_v.at[pl.ds(0, K)]], bufs[b], sin[b]).wait()

        def wb(c, b):
            pltpu.async_copy(
                bufs[b], out_hbm.at[pl.ds(base + c * K, K)], sout[b])

        def wb_wait(b):
            pltpu.make_async_copy(
                bufs[b], out_hbm.at[pl.ds(base, K)], sout[b]).wait()

        def step(c, b, wait_out, issue):
            gather_wait(b)
            wb(c, b)
            if wait_out:
                wb_wait((b + 2) % NBUF)
            if issue:
                gather(c + 2, (b + 2) % NBUF)

        gather(0, 0)
        gather(1, 1)
        step(0, 0, False, True)
        step(1, 1, True, True)
        step(2, 2, True, True)

        nrings = nchunk // NBUF

        @pl.loop(1, nrings)
        def ring(r):
            for b in range(NBUF):
                step(r * NBUF + b, b, True, True)

        step(nchunk - 2, (nchunk - 2) % NBUF, True, False)
        step(nchunk - 1, (nchunk - 1) % NBUF, False, False)
        wb_wait((nchunk - 2) % NBUF)
        wb_wait((nchunk - 1) % NBUF)

    return body(idx, table)


def kernel(x, W):
    B = x.size
    b_per_w = B // NW
    nchunk = b_per_w // K
    out = _emb_lookup(x.reshape(-1), W, b_per_w, nchunk)
    return out.reshape(x.shape + (W.shape[1],))


# speedup vs baseline: 1.7949x; 1.0202x over previous
import functools

import jax
import jax.numpy as jnp
from jax import lax
from jax.experimental import pallas as pl
from jax.experimental.pallas import tpu as pltpu
from jax.experimental.pallas import tpu_sc as plsc

NC = 2
NS = 16
NW = NC * NS

D = 2048
K = 8
NBUF = 3
NSB = 2


@functools.partial(jax.jit, static_argnums=(2, 3))
def _emb_lookup(idx, table, b_per_w, nchunk):
    mesh = plsc.VectorSubcoreMesh(
        core_axis_name="c", subcore_axis_name="s",
        num_cores=NC, num_subcores=NS,
    )
    B = NW * b_per_w

    @functools.partial(
        pl.kernel,
        out_type=jax.ShapeDtypeStruct((B, D), jnp.float32),
        mesh=mesh,
        scratch_types=[
            pltpu.VMEM((b_per_w,), jnp.int32),
            [pltpu.VMEM((K, D), jnp.float32)] * NBUF,
            pltpu.VMEM_SHARED((NS, NSB, K, D), jnp.float32),
            [pltpu.SemaphoreType.DMA] * NBUF,
            pltpu.SemaphoreType.DMA,
            [pltpu.SemaphoreType.DMA] * NSB,
        ],
    )
    def body(idx_hbm, table_hbm, out_hbm, idx_v, bufs, sbuf, sin, sm, sout):
        wid = lax.axis_index("s") * NC + lax.axis_index("c")
        sid = lax.axis_index("s")
        base = wid * b_per_w
        pltpu.sync_copy(idx_hbm.at[pl.ds(base, b_per_w)], idx_v)

        def gather(c, b):
            pltpu.async_copy(
                table_hbm.at[idx_v.at[pl.ds(c * K, K)]], bufs[b], sin[b])

        def gather_wait(b):
            pltpu.make_async_copy(
                table_hbm.at[idx_v.at[pl.ds(0, K)]], bufs[b], sin[b]).wait()

        def out_wait(m):
            pltpu.make_async_copy(
                sbuf.at[sid, m], out_hbm.at[pl.ds(base, K)], sout[m]).wait()

        def iteration(c, b6, wait_out, issue_in):
            b = b6 % NBUF
            m = b6 % NSB
            gather_wait(b)
            if wait_out:
                out_wait(m)
            pltpu.async_copy(bufs[b], sbuf.at[sid, m], sm)
            pltpu.make_async_copy(bufs[b], sbuf.at[sid, m], sm).wait()
            pltpu.async_copy(
                sbuf.at[sid, m], out_hbm.at[pl.ds(base + c * K, K)], sout[m])
            if issue_in:
                gather(c + NBUF, b)

        for b in range(NBUF):
            gather(b, b)
        for c in range(6):
            iteration(c, c, c >= NSB, True)

        nrings = nchunk // 6

        @pl.loop(1, nrings)
        def ring(r):
            for b6 in range(6):
                iteration(6 * r + b6, b6, True, True)

        for c in range(6 * nrings, nchunk):
            iteration(c, c % 6, True, c + NBUF < nchunk)
        out_wait((nchunk - 2) % NSB)
        out_wait((nchunk - 1) % NSB)

    return body(idx, table)


def kernel(x, W):
    B = x.size
    b_per_w = B // NW
    nchunk = b_per_w // K
    out = _emb_lookup(x.reshape(-1), W, b_per_w, nchunk)
    return out.reshape(x.shape + (W.shape[1],))
